# baseline (speedup 1.0000x reference)
import functools

import jax
import jax.numpy as jnp
from jax import lax
from jax.experimental import pallas as pl
from jax.experimental.pallas import tpu as pltpu
from jax.experimental.pallas import tpu_sc as plsc

C = 21
N = 512 * 512
NB = 1024
FQ = 16
CNT_SHIFT = 17
NTILES = 32
PPT = N // NTILES
CH = 1024
HROWS = 2 * C
HW = HROWS * NB


def _argmax_body(lab_ref, out_ref):
    x = lab_ref[...]
    v = x[0:1]
    idx = jnp.zeros_like(v, dtype=jnp.int32)
    for c in range(1, C):
        xc = x[c:c + 1]
        m = xc > v
        v = jnp.where(m, xc, v)
        idx = jnp.where(m, c, idx)
    out_ref[...] = idx[0]


def _argmax_labels(label):
    R = 32
    grid = 512 // R
    return pl.pallas_call(
        _argmax_body,
        grid=(grid,),
        in_specs=[pl.BlockSpec((C, R, 512), lambda i: (0, i, 0))],
        out_specs=pl.BlockSpec((R, 512), lambda i: (i, 0)),
        out_shape=jax.ShapeDtypeStruct((512, 512), jnp.int32),
    )(label)


def _sc_hist_kernel(pred_hbm, lbl_hbm, out_hbm, pred_v, lbl_v, hist_v,
                    sem_p0, sem_p1, sem_l0, sem_l1):
    wid = lax.axis_index("s") * 2 + lax.axis_index("c")
    base = wid * PPT

    @plsc.parallel_loop(0, HW // 16, 1, unroll=8)
    def zero_body(i):
        hist_v[pl.ds(i * 16, 16)] = jnp.zeros((16,), jnp.int32)

    nchunks = PPT // CH
    sems_p = (sem_p0, sem_p1)
    sems_l = (sem_l0, sem_l1)

    RPC = CH // 512
    row0 = wid * (PPT // 512)

    def start(k):
        s = k % 2
        hp = pltpu.async_copy(
            pred_hbm.at[:, pl.ds(row0 + k * RPC, RPC), :], pred_v.at[s],
            sems_p[s])
        hl = pltpu.async_copy(lbl_hbm.at[pl.ds(row0 + k * RPC, RPC), :],
                              lbl_v.at[s], sems_l[s])
        return hp, hl

    pending = start(0)
    for k in range(nchunks):
        s = k % 2
        pending[0].wait()
        pending[1].wait()
        if k + 1 < nchunks:
            pending = start(k + 1)

        NV = CH // 16

        @plsc.parallel_loop(0, C * NV, 1, unroll=8)
        def vec_body(i, s=s):
            c = i // NV
            v = i - c * NV
            r = v // 32
            col = (v - r * 32) * 16
            p = pred_v[s, c, r, pl.ds(col, 16)]
            q = (p * float(NB * FQ)).astype(jnp.int32)
            q = jnp.minimum(jnp.maximum(q, 0), NB * FQ - 1)
            val = (q & (FQ - 1)) + (1 << CNT_SHIFT)
            plsc.addupdate_scatter(
                hist_v, [(C + c) * NB + (q >> 4)], val.astype(jnp.int32))

        lanes = lax.broadcasted_iota(jnp.int32, (16,), 0)

        @plsc.parallel_loop(0, NV, 1, unroll=4)
        def fix_body(v, s=s):
            r = v // 32
            col = (v - r * 32) * 16
            k = lbl_v[s, r, pl.ds(col, 16)]
            pk = plsc.load_gather(
                pred_v,
                [jnp.full((16,), s, jnp.int32), k,
                 jnp.full((16,), r, jnp.int32), col + lanes])
            qb = (pk * float(NB * FQ)).astype(jnp.int32)
            qb = jnp.minimum(jnp.maximum(qb, 0), NB * FQ - 1)
            vb = (qb & (FQ - 1)) + (1 << CNT_SHIFT)
            plsc.addupdate_scatter(
                hist_v, [(C + k) * NB + (qb >> 4)], -vb)
            qf = ((1.0 - pk) * float(NB * FQ)).astype(jnp.int32)
            qf = jnp.minimum(jnp.maximum(qf, 0), NB * FQ - 1)
            vf = (qf & (FQ - 1)) + (1 << CNT_SHIFT)
            plsc.addupdate_scatter(
                hist_v, [k * NB + (qf >> 4)], vf.astype(jnp.int32))

    pltpu.sync_copy(hist_v, out_hbm.at[wid])


def _sc_hist(pred2, lbl):
    mesh = plsc.VectorSubcoreMesh(core_axis_name="c", subcore_axis_name="s")
    k = functools.partial(
        pl.kernel,
        out_type=jax.ShapeDtypeStruct((NTILES, HW), jnp.int32),
        mesh=mesh,
        compiler_params=pltpu.CompilerParams(needs_layout_passes=False),
        scratch_types=[
            pltpu.VMEM((2, C, CH // 512, 512), jnp.float32),
            pltpu.VMEM((2, CH // 512, 512), jnp.int32),
            pltpu.VMEM((HW,), jnp.int32),
            pltpu.SemaphoreType.DMA,
            pltpu.SemaphoreType.DMA,
            pltpu.SemaphoreType.DMA,
            pltpu.SemaphoreType.DMA,
        ],
    )(_sc_hist_kernel)
    return k(pred2, lbl)


def _final_body(hist_ref, out_ref):
    mask = (1 << CNT_SHIFT) - 1
    cnt_acc = jnp.zeros((HROWS, NB), jnp.float32)
    frac_acc = jnp.zeros((HROWS, NB), jnp.float32)
    for t in range(NTILES):
        h = hist_ref[t]
        cnt_acc = cnt_acc + (h >> CNT_SHIFT).astype(jnp.float32)
        frac_acc = frac_acc + (h & mask).astype(jnp.float32)
    if True:
        f = cnt_acc[0:C, :]
        g = cnt_acc[C:2 * C, :]
        fr = frac_acc[0:C, :] + frac_acc[C:2 * C, :]
        cnt_all = f + g
        bi = lax.broadcasted_iota(jnp.int32, (C, NB), 1).astype(jnp.float32)
        st = (bi * cnt_all + (fr + 0.5 * cnt_all) / float(FQ)) / float(NB)
        r = lax.broadcasted_iota(jnp.int32, (NB, NB), 0)
        cc = lax.broadcasted_iota(jnp.int32, (NB, NB), 1)
        T = (r <= cc).astype(jnp.float32)
        S_fg = jax.lax.dot(f, T, precision=jax.lax.Precision.HIGHEST)
        S_bg = jax.lax.dot(g, T, precision=jax.lax.Precision.HIGHEST)
        gts = jnp.sum(f, axis=1, keepdims=True)
        totbg = jnp.sum(g, axis=1, keepdims=True)
        den1 = jnp.maximum(gts + totbg - S_bg, 1.0)
        den2 = jnp.maximum(den1 + g, 1.0)
        dJ = S_fg / den1 - (S_fg - f) / den2
        ebar = st / jnp.maximum(cnt_all, 1.0)
        loss_main = jnp.sum(ebar * dJ, axis=1, keepdims=True)
        bif = lax.broadcasted_iota(jnp.int32, (C, NB), 1)
        bmax = jnp.max(jnp.where(cnt_all > 0, bif, -1), axis=1, keepdims=True)
        loss_empty = jnp.sum(jnp.where(bif == bmax, ebar, 0.0), axis=1,
                             keepdims=True)
        loss_c = jnp.where(gts == 0, loss_empty, loss_main)
        out_ref[...] = jnp.sum(loss_c, axis=0, keepdims=True) / float(C)


def _final_reduce(hist):
    return pl.pallas_call(
        _final_body,
        grid=(1,),
        in_specs=[pl.BlockSpec((NTILES, HROWS, NB), lambda i: (0, 0, 0))],
        out_specs=pl.BlockSpec((1, 1), lambda i: (0, 0)),
        out_shape=jax.ShapeDtypeStruct((1, 1), jnp.float32),
    )(hist)


def kernel(prediction, label):
    lbl = _argmax_labels(label)
    hist = _sc_hist(prediction, lbl)
    loss = _final_reduce(hist.reshape(NTILES, HROWS, NB))
    return loss.reshape(())


# speedup vs baseline: 1.0453x; 1.0453x over previous
import functools

import jax
import jax.numpy as jnp
from jax import lax
from jax.experimental import pallas as pl
from jax.experimental.pallas import tpu as pltpu
from jax.experimental.pallas import tpu_sc as plsc

C = 21
N = 512 * 512
NB = 1024
FQ = 16
CNT_SHIFT = 17
NTILES = 32
PPT = N // NTILES
CH = 1024
HROWS = 2 * C
HW = HROWS * NB


def _argmax_body(lab_ref, out_ref):
    x = lab_ref[...]
    v = x[0:1]
    idx = jnp.zeros_like(v, dtype=jnp.int32)
    for c in range(1, C):
        xc = x[c:c + 1]
        m = xc > v
        v = jnp.where(m, xc, v)
        idx = jnp.where(m, c, idx)
    out_ref[...] = idx[0]


def _argmax_labels(label):
    R = 32
    grid = 512 // R
    return pl.pallas_call(
        _argmax_body,
        grid=(grid,),
        in_specs=[pl.BlockSpec((C, R, 512), lambda i: (0, i, 0))],
        out_specs=pl.BlockSpec((R, 512), lambda i: (i, 0)),
        out_shape=jax.ShapeDtypeStruct((512, 512), jnp.int32),
    )(label)


def _sc_hist_kernel(pred_hbm, lbl_hbm, out_hbm, pred_v, lbl_v, hist_v,
                    sem_p0, sem_p1, sem_l0, sem_l1):
    wid = lax.axis_index("s") * 2 + lax.axis_index("c")
    base = wid * PPT

    @plsc.parallel_loop(0, HW // 16, 1, unroll=8)
    def zero_body(i):
        hist_v[pl.ds(i * 16, 16)] = jnp.zeros((16,), jnp.int32)

    nchunks = PPT // CH
    sems_p = (sem_p0, sem_p1)
    sems_l = (sem_l0, sem_l1)

    RPC = CH // 512
    row0 = wid * (PPT // 512)

    def start(k):
        s = k % 2
        hp = pltpu.async_copy(
            pred_hbm.at[:, pl.ds(row0 + k * RPC, RPC), :], pred_v.at[s],
            sems_p[s])
        hl = pltpu.async_copy(lbl_hbm.at[pl.ds(row0 + k * RPC, RPC), :],
                              lbl_v.at[s], sems_l[s])
        return hp, hl

    pending = start(0)
    for k in range(nchunks):
        s = k % 2
        pending[0].wait()
        pending[1].wait()
        if k + 1 < nchunks:
            pending = start(k + 1)

        NV = CH // 16

        @plsc.parallel_loop(0, C * NV, 1, unroll=8)
        def vec_body(i, s=s):
            c = i // NV
            v = i - c * NV
            r = v // 32
            col = (v - r * 32) * 16
            lblv = lbl_v[s, r, pl.ds(col, 16)]
            p = pred_v[s, c, r, pl.ds(col, 16)]
            m = lblv == c
            e = jnp.where(m, 1.0 - p, p)
            q = (e * float(NB * FQ)).astype(jnp.int32)
            q = jnp.minimum(jnp.maximum(q, 0), NB * FQ - 1)
            val = (q & (FQ - 1)) + (1 << CNT_SHIFT)
            rowbase = jnp.where(m, c * NB, (C + c) * NB)
            plsc.addupdate_scatter(
                hist_v, [rowbase + (q >> 4)], val.astype(jnp.int32))

    pltpu.sync_copy(hist_v, out_hbm.at[wid])


def _sc_hist(pred2, lbl):
    mesh = plsc.VectorSubcoreMesh(core_axis_name="c", subcore_axis_name="s")
    k = functools.partial(
        pl.kernel,
        out_type=jax.ShapeDtypeStruct((NTILES, HW), jnp.int32),
        mesh=mesh,
        compiler_params=pltpu.CompilerParams(needs_layout_passes=False),
        scratch_types=[
            pltpu.VMEM((2, C, CH // 512, 512), jnp.float32),
            pltpu.VMEM((2, CH // 512, 512), jnp.int32),
            pltpu.VMEM((HW,), jnp.int32),
            pltpu.SemaphoreType.DMA,
            pltpu.SemaphoreType.DMA,
            pltpu.SemaphoreType.DMA,
            pltpu.SemaphoreType.DMA,
        ],
    )(_sc_hist_kernel)
    return k(pred2, lbl)


def _final_body(hist_ref, out_ref):
    mask = (1 << CNT_SHIFT) - 1
    cnt_acc = jnp.zeros((HROWS, NB), jnp.float32)
    frac_acc = jnp.zeros((HROWS, NB), jnp.float32)
    for t in range(NTILES):
        h = hist_ref[t]
        cnt_acc = cnt_acc + (h >> CNT_SHIFT).astype(jnp.float32)
        frac_acc = frac_acc + (h & mask).astype(jnp.float32)
    if True:
        f = cnt_acc[0:C, :]
        g = cnt_acc[C:2 * C, :]
        fr = frac_acc[0:C, :] + frac_acc[C:2 * C, :]
        cnt_all = f + g
        bi = lax.broadcasted_iota(jnp.int32, (C, NB), 1).astype(jnp.float32)
        st = (bi * cnt_all + (fr + 0.5 * cnt_all) / float(FQ)) / float(NB)
        r = lax.broadcasted_iota(jnp.int32, (NB, NB), 0)
        cc = lax.broadcasted_iota(jnp.int32, (NB, NB), 1)
        T = (r <= cc).astype(jnp.float32)
        S_fg = jax.lax.dot(f, T, precision=jax.lax.Precision.HIGHEST)
        S_bg = jax.lax.dot(g, T, precision=jax.lax.Precision.HIGHEST)
        gts = jnp.sum(f, axis=1, keepdims=True)
        totbg = jnp.sum(g, axis=1, keepdims=True)
        den1 = jnp.maximum(gts + totbg - S_bg, 1.0)
        den2 = jnp.maximum(den1 + g, 1.0)
        dJ = S_fg / den1 - (S_fg - f) / den2
        ebar = st / jnp.maximum(cnt_all, 1.0)
        loss_main = jnp.sum(ebar * dJ, axis=1, keepdims=True)
        bif = lax.broadcasted_iota(jnp.int32, (C, NB), 1)
        bmax = jnp.max(jnp.where(cnt_all > 0, bif, -1), axis=1, keepdims=True)
        loss_empty = jnp.sum(jnp.where(bif == bmax, ebar, 0.0), axis=1,
                             keepdims=True)
        loss_c = jnp.where(gts == 0, loss_empty, loss_main)
        out_ref[...] = jnp.sum(loss_c, axis=0, keepdims=True) / float(C)


def _final_reduce(hist):
    return pl.pallas_call(
        _final_body,
        grid=(1,),
        in_specs=[pl.BlockSpec((NTILES, HROWS, NB), lambda i: (0, 0, 0))],
        out_specs=pl.BlockSpec((1, 1), lambda i: (0, 0)),
        out_shape=jax.ShapeDtypeStruct((1, 1), jnp.float32),
    )(hist)


def kernel(prediction, label):
    lbl = _argmax_labels(label)
    hist = _sc_hist(prediction, lbl)
    loss = _final_reduce(hist.reshape(NTILES, HROWS, NB))
    return loss.reshape(())


# speedup vs baseline: 1.0464x; 1.0010x over previous
import functools

import jax
import jax.numpy as jnp
from jax import lax
from jax.experimental import pallas as pl
from jax.experimental.pallas import tpu as pltpu
from jax.experimental.pallas import tpu_sc as plsc

C = 21
N = 512 * 512
NB = 1024
FQ = 16
CNT_SHIFT = 17
NTILES = 32
PPT = N // NTILES
CH = 1024
HROWS = 2 * C
HW = HROWS * NB


def _argmax_body(lab_ref, out_ref):
    x = lab_ref[...]
    v = x[0:1]
    idx = jnp.zeros_like(v, dtype=jnp.int32)
    for c in range(1, C):
        xc = x[c:c + 1]
        m = xc > v
        v = jnp.where(m, xc, v)
        idx = jnp.where(m, c, idx)
    out_ref[...] = idx[0]


def _argmax_labels(label):
    R = 32
    grid = 512 // R
    return pl.pallas_call(
        _argmax_body,
        grid=(grid,),
        in_specs=[pl.BlockSpec((C, R, 512), lambda i: (0, i, 0))],
        out_specs=pl.BlockSpec((R, 512), lambda i: (i, 0)),
        out_shape=jax.ShapeDtypeStruct((512, 512), jnp.int32),
    )(label)


def _sc_hist_kernel(pred_hbm, lbl_hbm, out_hbm, pred_v, lbl_v, hist_v,
                    sem_p0, sem_p1, sem_l0, sem_l1):
    wid = lax.axis_index("s") * 2 + lax.axis_index("c")
    base = wid * PPT

    @plsc.parallel_loop(0, HW // 16, 1, unroll=8)
    def zero_body(i):
        hist_v[pl.ds(i * 16, 16)] = jnp.zeros((16,), jnp.int32)

    nchunks = PPT // CH
    sems_p = (sem_p0, sem_p1)
    sems_l = (sem_l0, sem_l1)

    RPC = CH // 512
    row0 = wid * (PPT // 512)

    def start(k):
        s = k % 2
        hp = pltpu.async_copy(
            pred_hbm.at[:, pl.ds(row0 + k * RPC, RPC), :], pred_v.at[s],
            sems_p[s])
        hl = pltpu.async_copy(lbl_hbm.at[pl.ds(row0 + k * RPC, RPC), :],
                              lbl_v.at[s], sems_l[s])
        return hp, hl

    pending = start(0)
    for k in range(nchunks):
        s = k % 2
        pending[0].wait()
        pending[1].wait()
        if k + 1 < nchunks:
            pending = start(k + 1)

        NV = CH // 16

        @plsc.parallel_loop(0, C * NV, 1, unroll=8)
        def vec_body(i, s=s):
            c = i // NV
            v = i - c * NV
            r = v // 32
            col = (v - r * 32) * 16
            lblv = lbl_v[s, r, pl.ds(col, 16)]
            p = pred_v[s, c, r, pl.ds(col, 16)]
            m = lblv == c
            e = jnp.where(m, 1.0 - p, p)
            q = (e * float(NB * FQ)).astype(jnp.int32)
            q = jnp.minimum(jnp.maximum(q, 0), NB * FQ - 1)
            val = (q & (FQ - 1)) + (1 << CNT_SHIFT)
            rowbase = jnp.where(m, c * NB, (C + c) * NB)
            plsc.addupdate_scatter(
                hist_v, [rowbase + (q >> 4)], val.astype(jnp.int32))

    pltpu.sync_copy(hist_v, out_hbm.at[wid])


def _sc_hist(pred2, lbl):
    mesh = plsc.VectorSubcoreMesh(core_axis_name="c", subcore_axis_name="s")
    k = functools.partial(
        pl.kernel,
        out_type=jax.ShapeDtypeStruct((NTILES, HW), jnp.int32),
        mesh=mesh,
        compiler_params=pltpu.CompilerParams(needs_layout_passes=False),
        scratch_types=[
            pltpu.VMEM((2, C, CH // 512, 512), jnp.float32),
            pltpu.VMEM((2, CH // 512, 512), jnp.int32),
            pltpu.VMEM((HW,), jnp.int32),
            pltpu.SemaphoreType.DMA,
            pltpu.SemaphoreType.DMA,
            pltpu.SemaphoreType.DMA,
            pltpu.SemaphoreType.DMA,
        ],
    )(_sc_hist_kernel)
    return k(pred2, lbl)


def _final_body(hist_ref, out_ref):
    mask = (1 << CNT_SHIFT) - 1
    cnt_acc = jnp.zeros((HROWS, NB), jnp.float32)
    frac_acc = jnp.zeros((HROWS, NB), jnp.float32)
    for t in range(NTILES):
        h = hist_ref[t]
        cnt_acc = cnt_acc + (h >> CNT_SHIFT).astype(jnp.float32)
        frac_acc = frac_acc + (h & mask).astype(jnp.float32)
    f = cnt_acc[0:C, :]
    g = cnt_acc[C:2 * C, :]
    fr = frac_acc[0:C, :] + frac_acc[C:2 * C, :]
    cnt_all = f + g
    bi = lax.broadcasted_iota(jnp.int32, (C, NB), 1).astype(jnp.float32)
    st = (bi * cnt_all + (fr + 0.5 * cnt_all) / float(FQ)) / float(NB)
    r = lax.broadcasted_iota(jnp.int32, (NB, NB), 0)
    cc = lax.broadcasted_iota(jnp.int32, (NB, NB), 1)
    T = (r <= cc).astype(jnp.float32)
    S_fg = jax.lax.dot(f, T, precision=jax.lax.Precision.HIGHEST)
    S_bg = jax.lax.dot(g, T, precision=jax.lax.Precision.HIGHEST)
    gts = jnp.sum(f, axis=1, keepdims=True)
    totbg = jnp.sum(g, axis=1, keepdims=True)
    den1 = jnp.maximum(gts + totbg - S_bg, 1.0)
    den2 = jnp.maximum(den1 + g, 1.0)
    dJ = S_fg / den1 - (S_fg - f) / den2
    ebar = st / jnp.maximum(cnt_all, 1.0)
    loss_main = jnp.sum(ebar * dJ, axis=1, keepdims=True)
    bif = lax.broadcasted_iota(jnp.int32, (C, NB), 1)
    bmax = jnp.max(jnp.where(cnt_all > 0, bif, -1), axis=1, keepdims=True)
    loss_empty = jnp.sum(jnp.where(bif == bmax, ebar, 0.0), axis=1,
                         keepdims=True)
    loss_c = jnp.where(gts == 0, loss_empty, loss_main)
    out_ref[...] = jnp.sum(loss_c, axis=0, keepdims=True) / float(C)


def _final_reduce(hist):
    return pl.pallas_call(
        _final_body,
        grid=(1,),
        in_specs=[pl.BlockSpec((NTILES, HROWS, NB), lambda i: (0, 0, 0))],
        out_specs=pl.BlockSpec((1, 1), lambda i: (0, 0)),
        out_shape=jax.ShapeDtypeStruct((1, 1), jnp.float32),
    )(hist)


def kernel(prediction, label):
    lbl = _argmax_labels(label)
    hist = _sc_hist(prediction, lbl)
    loss = _final_reduce(hist.reshape(NTILES, HROWS, NB))
    return loss.reshape(())
